# baseline (speedup 1.0000x reference)
import functools

import jax
import jax.numpy as jnp
from jax import lax
from jax.experimental import pallas as pl
from jax.experimental.pallas import tpu as pltpu
from jax.experimental.pallas import tpu_sc as plsc

N = 10000
E = 320000
D = 128
DE = 16
DT = 100
DTP = 128
H = 2
DH = 64
NP = 10240
ACC_W = 144

NC = 2
NS = 16
NW = NC * NS
NSPLIT = 10
EH = E // NSPLIT
EWH = EH // NW
GC = 40
ROWS_PER_TILE = NP // NS

BE = 1280
BN = 128


def _node_proj_body(h_ref, wqh_ref, wqt_ref, te0_ref, bq_ref, wkv_ref,
                    q_ref, kv_ref):
    h = h_ref[...]
    qb = te0_ref[...] @ wqt_ref[...] + bq_ref[...]
    q_ref[...] = h @ wqh_ref[...] + qb
    kv_ref[...] = h @ wkv_ref[...]


def _node_proj(h, wqh, wqt, te0, bq, wkv):
    grid = NP // BN
    return pl.pallas_call(
        _node_proj_body,
        grid=(grid,),
        in_specs=[
            pl.BlockSpec((BN, D), lambda i: (i, 0)),
            pl.BlockSpec((D, D), lambda i: (0, 0)),
            pl.BlockSpec((DTP, D), lambda i: (0, 0)),
            pl.BlockSpec((1, DTP), lambda i: (0, 0)),
            pl.BlockSpec((1, D), lambda i: (0, 0)),
            pl.BlockSpec((D, 2 * D), lambda i: (0, 0)),
        ],
        out_specs=[
            pl.BlockSpec((BN, D), lambda i: (i, 0)),
            pl.BlockSpec((BN, 2 * D), lambda i: (i, 0)),
        ],
        out_shape=[
            jax.ShapeDtypeStruct((NP, D), jnp.float32),
            jax.ShapeDtypeStruct((NP, 2 * D), jnp.float32),
        ],
    )(h, wqh, wqt, te0, bq, wkv)


def _edge_msg_body(t_ref, ef_ref, qg_ref, kvg_ref, tw_ref, tb_ref,
                   wt2_ref, we2_ref, bkv_ref, msg_ref, w_ref):
    tcols = t_ref[0]
    tfull = jnp.concatenate(
        [jnp.broadcast_to(tcols[:, j:j + 1], (128, DTP))
         for j in range(BE // 128)], axis=0)
    te = jnp.cos(tfull * tw_ref[...] + tb_ref[...])
    kv = kvg_ref[...] + te @ wt2_ref[...] + ef_ref[...] @ we2_ref[...] \
        + bkv_ref[...]
    k = kv[:, :D]
    v = kv[:, D:]
    qk = qg_ref[...] * k
    s0 = jnp.sum(qk[:, :DH], axis=1, keepdims=True) * (1.0 / 8.0)
    s1 = jnp.sum(qk[:, DH:], axis=1, keepdims=True) * (1.0 / 8.0)
    w0 = jnp.exp(s0)
    w1 = jnp.exp(s1)
    msg_ref[:, 0:DH] = v[:, :DH] * w0
    msg_ref[:, DH:D] = v[:, DH:] * w1
    w_ref[:, 0:1] = w0
    w_ref[:, 1:2] = w1


def _edge_msg(t2, ef, qg, kvg, tw, tb, wt2, we2, bkv):
    grid = EH // BE
    return pl.pallas_call(
        _edge_msg_body,
        grid=(grid,),
        in_specs=[
            pl.BlockSpec((1, 128, BE // 128), lambda i: (i, 0, 0)),
            pl.BlockSpec((BE, DE), lambda i: (i, 0)),
            pl.BlockSpec((BE, D), lambda i: (i, 0)),
            pl.BlockSpec((BE, 2 * D), lambda i: (i, 0)),
            pl.BlockSpec((1, DTP), lambda i: (0, 0)),
            pl.BlockSpec((1, DTP), lambda i: (0, 0)),
            pl.BlockSpec((DTP, 2 * D), lambda i: (0, 0)),
            pl.BlockSpec((DE, 2 * D), lambda i: (0, 0)),
            pl.BlockSpec((1, 2 * D), lambda i: (0, 0)),
        ],
        out_specs=[
            pl.BlockSpec((BE, D), lambda i: (i, 0)),
            pl.BlockSpec((BE, D), lambda i: (i, 0)),
        ],
        out_shape=[
            jax.ShapeDtypeStruct((EH, D), jnp.float32),
            jax.ShapeDtypeStruct((EH, D), jnp.float32),
        ],
    )(t2, ef, qg, kvg, tw, tb, wt2, we2, bkv)


def _out_proj_body(h_ref, *rest):
    p_refs = rest[:NSPLIT]
    woh_ref, woa_ref, bo_ref, g_ref, b_ref, o_ref = rest[NSPLIT:]
    acc = p_refs[0][0, 0] + p_refs[0][1, 0]
    wv = p_refs[0][0, 1] + p_refs[0][1, 1]
    for pr in p_refs[1:]:
        acc = acc + pr[0, 0] + pr[1, 0]
        wv = wv + pr[0, 1] + pr[1, 1]
    w0 = jnp.maximum(wv[:, 0:1], 1e-30)
    w1 = jnp.maximum(wv[:, 1:2], 1e-30)
    agg = jnp.concatenate([acc[:, 0:DH] / w0, acc[:, DH:D] / w1], axis=1)
    out = h_ref[...] @ woh_ref[...] + agg @ woa_ref[...] + bo_ref[...]
    out = jnp.maximum(out, 0.0)
    mu = jnp.mean(out, axis=1, keepdims=True)
    var = jnp.mean((out - mu) * (out - mu), axis=1, keepdims=True)
    o_ref[...] = (out - mu) * lax.rsqrt(var + 1e-5) * g_ref[...] + b_ref[...]


def _out_proj(h, parts, woh, woa, bo, g, b):
    grid = NP // BN
    return pl.pallas_call(
        _out_proj_body,
        grid=(grid,),
        in_specs=[
            pl.BlockSpec((BN, D), lambda i: (i, 0)),
        ] + [
            pl.BlockSpec((2, 2, BN, D), lambda i: (0, 0, i, 0))
            for _ in range(NSPLIT)
        ] + [
            pl.BlockSpec((D, D), lambda i: (0, 0)),
            pl.BlockSpec((D, D), lambda i: (0, 0)),
            pl.BlockSpec((1, D), lambda i: (0, 0)),
            pl.BlockSpec((1, D), lambda i: (0, 0)),
            pl.BlockSpec((1, D), lambda i: (0, 0)),
        ],
        out_specs=pl.BlockSpec((BN, D), lambda i: (i, 0)),
        out_shape=jax.ShapeDtypeStruct((NP, D), jnp.float32),
    )(h, *parts, woh, woa, bo, g, b)


def _link_score_body(hs_ref, x_ref, wsrc_ref, wdst_ref, b_ref, wp_ref,
                     bp_ref, o_ref):
    z = hs_ref[...] @ wsrc_ref[...] + x_ref[0] @ wdst_ref[...] + b_ref[...]
    z = jnp.maximum(z, 0.0)
    o_ref[...] = z @ wp_ref[...] + bp_ref[...]


def _link_score(hs, hdn, wsrc, wdst, b, wp, bp):
    B = hs.shape[0]
    return pl.pallas_call(
        _link_score_body,
        grid=(2,),
        in_specs=[
            pl.BlockSpec((B, D), lambda i: (0, 0)),
            pl.BlockSpec((1, B, D), lambda i: (i, 0, 0)),
            pl.BlockSpec((D, D), lambda i: (0, 0)),
            pl.BlockSpec((D, D), lambda i: (0, 0)),
            pl.BlockSpec((1, D), lambda i: (0, 0)),
            pl.BlockSpec((D, 1), lambda i: (0, 0)),
            pl.BlockSpec((1, 1), lambda i: (0, 0)),
        ],
        out_specs=pl.BlockSpec((B, 1), lambda i: (i, 0)),
        out_shape=jax.ShapeDtypeStruct((2 * B, 1), jnp.float32),
    )(hs, hdn, wsrc, wdst, b, wp, bp)


def _sc_mesh():
    return plsc.VectorSubcoreMesh(core_axis_name="c", subcore_axis_name="s")


def _sc_gather_qkv(qn, kv, src, dst):
    niter = EWH // GC

    @functools.partial(
        pl.kernel,
        out_type=(jax.ShapeDtypeStruct((EH, D), jnp.float32),
                  jax.ShapeDtypeStruct((EH, 2 * D), jnp.float32)),
        mesh=_sc_mesh(),
        scratch_types=[
            pltpu.VMEM((GC,), jnp.int32),
            pltpu.VMEM((GC,), jnp.int32),
            pltpu.VMEM((GC,), jnp.int32),
            pltpu.VMEM((GC,), jnp.int32),
            pltpu.VMEM((GC, D), jnp.float32),
            pltpu.VMEM((GC, D), jnp.float32),
            pltpu.VMEM((GC, 2 * D), jnp.float32),
            pltpu.VMEM((GC, 2 * D), jnp.float32),
            pltpu.SemaphoreType.DMA,
            pltpu.SemaphoreType.DMA,
            pltpu.SemaphoreType.DMA,
            pltpu.SemaphoreType.DMA,
            pltpu.SemaphoreType.DMA,
            pltpu.SemaphoreType.DMA,
        ],
    )
    def k(qn_h, kv_h, src_h, dst_h, qg_h, kvg_h,
          di0, di1, si0, si1, qb0, qb1, kb0, kb1,
          smi0, smi1, sg0, sg1, sw0, sw1):
        wid = lax.axis_index("s") * NC + lax.axis_index("c")
        base = wid * EWH
        dis, sis = (di0, di1), (si0, si1)
        qbs, kbs = (qb0, qb1), (kb0, kb1)
        smis, sgs, sws = (smi0, smi1), (sg0, sg1), (sw0, sw1)
        ih = [None, None]
        g = [None, None]
        w = [None, None]
        for i in range(niter + 1):
            p = i & 1
            q = 1 - p
            if i < niter:
                if g[p] is not None:
                    g[p][0].wait()
                    g[p][1].wait()
                    off2 = base + (i - 2) * GC
                    w[p] = (pltpu.async_copy(qbs[p],
                                             qg_h.at[pl.ds(off2, GC)],
                                             sws[p]),
                            pltpu.async_copy(kbs[p],
                                             kvg_h.at[pl.ds(off2, GC)],
                                             sws[p]))
                    g[p] = None
                off = base + i * GC
                ih[p] = (pltpu.async_copy(dst_h.at[pl.ds(off, GC)], dis[p],
                                          smis[p]),
                         pltpu.async_copy(src_h.at[pl.ds(off, GC)], sis[p],
                                          smis[p]))
            if i > 0:
                ih[q][0].wait()
                ih[q][1].wait()
                if w[q] is not None:
                    w[q][0].wait()
                    w[q][1].wait()
                    w[q] = None
                g[q] = (pltpu.async_copy(qn_h.at[dis[q]], qbs[q], sgs[q]),
                        pltpu.async_copy(kv_h.at[sis[q]], kbs[q], sgs[q]))
        for c in (niter - 2, niter - 1):
            pc = c & 1
            if g[pc] is not None:
                g[pc][0].wait()
                g[pc][1].wait()
                if w[pc] is not None:
                    w[pc][0].wait()
                    w[pc][1].wait()
                offc = base + c * GC
                w[pc] = (pltpu.async_copy(qbs[pc],
                                          qg_h.at[pl.ds(offc, GC)], sws[pc]),
                         pltpu.async_copy(kbs[pc],
                                          kvg_h.at[pl.ds(offc, GC)],
                                          sws[pc]))
        for pc in (0, 1):
            if w[pc] is not None:
                w[pc][0].wait()
                w[pc][1].wait()

    return k(qn, kv, src, dst)


def _sc_scatter_msg(msg, wrow, dst, zacc):
    niter = EWH // GC

    @functools.partial(
        pl.kernel,
        out_type=jax.ShapeDtypeStruct((NC, 2, NP, D), jnp.float32),
        mesh=_sc_mesh(),
        scratch_types=[
            pltpu.VMEM((GC,), jnp.int32),
            pltpu.VMEM((GC,), jnp.int32),
            pltpu.VMEM((GC, D), jnp.float32),
            pltpu.VMEM((GC, D), jnp.float32),
            pltpu.VMEM_SHARED((NP, D), jnp.float32),
            pltpu.SemaphoreType.DMA,
            pltpu.SemaphoreType.DMA,
            pltpu.SemaphoreType.DMA,
            pltpu.SemaphoreType.DMA,
        ],
    )
    def k(msg_h, w_h, dst_h, zacc_h, out_h, di0, di1, mb0, mb1, acc_s,
          sr0, sr1, sa0, sa1):
        cid = lax.axis_index("c")
        sid = lax.axis_index("s")
        wid = sid * NC + cid
        base = wid * EWH
        r0 = sid * ROWS_PER_TILE
        dis, mbs, srs, sas = (di0, di1), (mb0, mb1), (sr0, sr1), (sa0, sa1)

        for phase, src_h in ((0, msg_h), (1, w_h)):
            pltpu.sync_copy(zacc_h.at[pl.ds(r0, ROWS_PER_TILE)],
                            acc_s.at[pl.ds(r0, ROWS_PER_TILE)])
            plsc.subcore_barrier()
            r = [None, None]
            a = [None, None]
            for i in range(niter):
                p = i & 1
                if a[p] is not None:
                    a[p].wait()
                off = base + i * GC
                r[p] = (pltpu.async_copy(dst_h.at[pl.ds(off, GC)], dis[p],
                                         srs[p]),
                        pltpu.async_copy(src_h.at[pl.ds(off, GC)], mbs[p],
                                         srs[p]))
                if i > 0:
                    q = 1 - p
                    r[q][0].wait()
                    r[q][1].wait()
                    a[q] = pltpu.async_copy(mbs[q], acc_s.at[dis[q]],
                                            sas[q], add=True)
            p = (niter - 1) & 1
            r[p][0].wait()
            r[p][1].wait()
            al = pltpu.async_copy(mbs[p], acc_s.at[dis[p]], sas[p], add=True)
            if a[1 - p] is not None:
                a[1 - p].wait()
            al.wait()
            plsc.subcore_barrier()
            pltpu.sync_copy(acc_s.at[pl.ds(r0, ROWS_PER_TILE)],
                            out_h.at[cid, phase, pl.ds(r0, ROWS_PER_TILE)])
            plsc.subcore_barrier()

    return k(msg, wrow, dst, zacc)


def _sc_gather_rows(table, ids):
    n = ids.shape[0]
    c = n // NW

    @functools.partial(
        pl.kernel,
        out_type=jax.ShapeDtypeStruct((n, D), jnp.float32),
        mesh=_sc_mesh(),
        scratch_types=[
            pltpu.VMEM((c,), jnp.int32),
            pltpu.VMEM((c, D), jnp.float32),
            pltpu.SemaphoreType.DMA,
        ],
    )
    def k(tab_h, ids_h, out_h, idx, buf, sem):
        wid = lax.axis_index("s") * NC + lax.axis_index("c")
        base = wid * c
        pltpu.sync_copy(ids_h.at[pl.ds(base, c)], idx)
        pltpu.async_copy(tab_h.at[idx], buf, sem).wait()
        pltpu.sync_copy(buf, out_h.at[pl.ds(base, c)])

    return k(table, ids)


def _pad_rows(x, rows):
    return jnp.pad(x, ((0, rows - x.shape[0]), (0, 0)))


def kernel(node_feat, edge_index, edge_feat, edge_time, src_ids, dst_ids,
           neg_ids, time_w, time_b, Wq, bq, Wk, bk, Wv, bv, Wout, bout,
           gamma, beta, Wsrc, bsrc, Wdst, bdst, Wp, bp):
    f32 = jnp.float32
    srcs = [edge_index[0, s * EH:(s + 1) * EH] for s in range(NSPLIT)]
    dsts = [edge_index[1, s * EH:(s + 1) * EH] for s in range(NSPLIT)]
    efs = [edge_feat[s * EH:(s + 1) * EH] for s in range(NSPLIT)]
    t2s = [edge_time[s * EH:(s + 1) * EH].astype(f32)
           .reshape(EH // BE, BE // 128, 128).swapaxes(1, 2)
           for s in range(NSPLIT)]
    h = _pad_rows(node_feat.astype(f32), NP)
    zacc = jnp.zeros((NP, D), f32)

    for l in range(2):
        wqh = Wq[l][:D]
        wqt = _pad_rows(Wq[l][D:D + DT], DTP)
        te0 = jnp.pad(jnp.cos(time_b[l]), (0, DTP - DT)).reshape(1, DTP)
        bq_row = bq[l].reshape(1, D)
        wkv = jnp.concatenate([Wk[l][:D], Wv[l][:D]], axis=1)
        wt2 = _pad_rows(
            jnp.concatenate([Wk[l][D + DE:], Wv[l][D + DE:]], axis=1), DTP)
        we2 = jnp.concatenate([Wk[l][D:D + DE], Wv[l][D:D + DE]], axis=1)
        bkv = jnp.concatenate([bk[l], bv[l]]).reshape(1, 2 * D)
        tw_row = jnp.pad(time_w[l], (0, DTP - DT)).reshape(1, DTP)
        tb_row = jnp.pad(time_b[l], (0, DTP - DT)).reshape(1, DTP)

        qn, kvn = _node_proj(h, wqh, wqt, te0, bq_row, wkv)
        parts = []
        for s in range(NSPLIT):
            qg, kvg = _sc_gather_qkv(qn, kvn, srcs[s], dsts[s])
            msg, wrow = _edge_msg(t2s[s], efs[s], qg, kvg, tw_row, tb_row,
                                  wt2, we2, bkv)
            parts.append(_sc_scatter_msg(msg, wrow, dsts[s], zacc))
        h = _out_proj(h, parts, Wout[l][:D], Wout[l][D:],
                      bout[l].reshape(1, D), gamma[l].reshape(1, D),
                      beta[l].reshape(1, D))

    ids = jnp.concatenate([src_ids, dst_ids, neg_ids]).astype(jnp.int32)
    g = _sc_gather_rows(h, ids)
    B = src_ids.shape[0]
    hs = g[:B]
    hdn = g[B:].reshape(2, B, D)
    bb = (bsrc + bdst).reshape(1, D)
    return _link_score(hs, hdn, Wsrc, Wdst, bb, Wp, bp.reshape(1, 1))


# speedup vs baseline: 1.0015x; 1.0015x over previous
import functools

import jax
import jax.numpy as jnp
from jax import lax
from jax.experimental import pallas as pl
from jax.experimental.pallas import tpu as pltpu
from jax.experimental.pallas import tpu_sc as plsc

N = 10000
E = 320000
D = 128
DE = 16
DT = 100
DTP = 128
H = 2
DH = 64
NP = 10240
ACC_W = 144

NC = 2
NS = 16
NW = NC * NS
NSPLIT = 5
EH = E // NSPLIT
EWH = EH // NW
GC = 40
ROWS_PER_TILE = NP // NS

BE = 1280
BN = 128


def _node_proj_body(h_ref, wqh_ref, wqt_ref, te0_ref, bq_ref, wkv_ref,
                    q_ref, kv_ref):
    h = h_ref[...]
    qb = te0_ref[...] @ wqt_ref[...] + bq_ref[...]
    q_ref[...] = h @ wqh_ref[...] + qb
    kv_ref[...] = h @ wkv_ref[...]


def _node_proj(h, wqh, wqt, te0, bq, wkv):
    grid = NP // BN
    return pl.pallas_call(
        _node_proj_body,
        grid=(grid,),
        in_specs=[
            pl.BlockSpec((BN, D), lambda i: (i, 0)),
            pl.BlockSpec((D, D), lambda i: (0, 0)),
            pl.BlockSpec((DTP, D), lambda i: (0, 0)),
            pl.BlockSpec((1, DTP), lambda i: (0, 0)),
            pl.BlockSpec((1, D), lambda i: (0, 0)),
            pl.BlockSpec((D, 2 * D), lambda i: (0, 0)),
        ],
        out_specs=[
            pl.BlockSpec((BN, D), lambda i: (i, 0)),
            pl.BlockSpec((BN, 2 * D), lambda i: (i, 0)),
        ],
        out_shape=[
            jax.ShapeDtypeStruct((NP, D), jnp.float32),
            jax.ShapeDtypeStruct((NP, 2 * D), jnp.float32),
        ],
    )(h, wqh, wqt, te0, bq, wkv)


def _edge_msg_body(t_ref, ef_ref, qg_ref, kvg_ref, tw_ref, tb_ref,
                   wt2_ref, we2_ref, bkv_ref, msg_ref, w_ref):
    tcols = t_ref[0]
    tfull = jnp.concatenate(
        [jnp.broadcast_to(tcols[:, j:j + 1], (128, DTP))
         for j in range(BE // 128)], axis=0)
    te = jnp.cos(tfull * tw_ref[...] + tb_ref[...])
    kv = kvg_ref[...] + te @ wt2_ref[...] + ef_ref[...] @ we2_ref[...] \
        + bkv_ref[...]
    k = kv[:, :D]
    v = kv[:, D:]
    qk = qg_ref[...] * k
    s0 = jnp.sum(qk[:, :DH], axis=1, keepdims=True) * (1.0 / 8.0)
    s1 = jnp.sum(qk[:, DH:], axis=1, keepdims=True) * (1.0 / 8.0)
    w0 = jnp.exp(s0)
    w1 = jnp.exp(s1)
    msg_ref[:, 0:DH] = v[:, :DH] * w0
    msg_ref[:, DH:D] = v[:, DH:] * w1
    w_ref[:, 0:1] = w0
    w_ref[:, 1:2] = w1


def _edge_msg(t2, ef, qg, kvg, tw, tb, wt2, we2, bkv):
    grid = EH // BE
    return pl.pallas_call(
        _edge_msg_body,
        grid=(grid,),
        in_specs=[
            pl.BlockSpec((1, 128, BE // 128), lambda i: (i, 0, 0)),
            pl.BlockSpec((BE, DE), lambda i: (i, 0)),
            pl.BlockSpec((BE, D), lambda i: (i, 0)),
            pl.BlockSpec((BE, 2 * D), lambda i: (i, 0)),
            pl.BlockSpec((1, DTP), lambda i: (0, 0)),
            pl.BlockSpec((1, DTP), lambda i: (0, 0)),
            pl.BlockSpec((DTP, 2 * D), lambda i: (0, 0)),
            pl.BlockSpec((DE, 2 * D), lambda i: (0, 0)),
            pl.BlockSpec((1, 2 * D), lambda i: (0, 0)),
        ],
        out_specs=[
            pl.BlockSpec((BE, D), lambda i: (i, 0)),
            pl.BlockSpec((BE, D), lambda i: (i, 0)),
        ],
        out_shape=[
            jax.ShapeDtypeStruct((EH, D), jnp.float32),
            jax.ShapeDtypeStruct((EH, D), jnp.float32),
        ],
    )(t2, ef, qg, kvg, tw, tb, wt2, we2, bkv)


def _out_proj_body(h_ref, *rest):
    p_refs = rest[:NSPLIT]
    woh_ref, woa_ref, bo_ref, g_ref, b_ref, o_ref = rest[NSPLIT:]
    acc = p_refs[0][0, 0] + p_refs[0][1, 0]
    wv = p_refs[0][0, 1] + p_refs[0][1, 1]
    for pr in p_refs[1:]:
        acc = acc + pr[0, 0] + pr[1, 0]
        wv = wv + pr[0, 1] + pr[1, 1]
    w0 = jnp.maximum(wv[:, 0:1], 1e-30)
    w1 = jnp.maximum(wv[:, 1:2], 1e-30)
    agg = jnp.concatenate([acc[:, 0:DH] / w0, acc[:, DH:D] / w1], axis=1)
    out = h_ref[...] @ woh_ref[...] + agg @ woa_ref[...] + bo_ref[...]
    out = jnp.maximum(out, 0.0)
    mu = jnp.mean(out, axis=1, keepdims=True)
    var = jnp.mean((out - mu) * (out - mu), axis=1, keepdims=True)
    o_ref[...] = (out - mu) * lax.rsqrt(var + 1e-5) * g_ref[...] + b_ref[...]


def _out_proj(h, parts, woh, woa, bo, g, b):
    grid = NP // BN
    return pl.pallas_call(
        _out_proj_body,
        grid=(grid,),
        in_specs=[
            pl.BlockSpec((BN, D), lambda i: (i, 0)),
        ] + [
            pl.BlockSpec((2, 2, BN, D), lambda i: (0, 0, i, 0))
            for _ in range(NSPLIT)
        ] + [
            pl.BlockSpec((D, D), lambda i: (0, 0)),
            pl.BlockSpec((D, D), lambda i: (0, 0)),
            pl.BlockSpec((1, D), lambda i: (0, 0)),
            pl.BlockSpec((1, D), lambda i: (0, 0)),
            pl.BlockSpec((1, D), lambda i: (0, 0)),
        ],
        out_specs=pl.BlockSpec((BN, D), lambda i: (i, 0)),
        out_shape=jax.ShapeDtypeStruct((NP, D), jnp.float32),
    )(h, *parts, woh, woa, bo, g, b)


def _link_score_body(hs_ref, x_ref, wsrc_ref, wdst_ref, b_ref, wp_ref,
                     bp_ref, o_ref):
    z = hs_ref[...] @ wsrc_ref[...] + x_ref[0] @ wdst_ref[...] + b_ref[...]
    z = jnp.maximum(z, 0.0)
    o_ref[...] = z @ wp_ref[...] + bp_ref[...]


def _link_score(hs, hdn, wsrc, wdst, b, wp, bp):
    B = hs.shape[0]
    return pl.pallas_call(
        _link_score_body,
        grid=(2,),
        in_specs=[
            pl.BlockSpec((B, D), lambda i: (0, 0)),
            pl.BlockSpec((1, B, D), lambda i: (i, 0, 0)),
            pl.BlockSpec((D, D), lambda i: (0, 0)),
            pl.BlockSpec((D, D), lambda i: (0, 0)),
            pl.BlockSpec((1, D), lambda i: (0, 0)),
            pl.BlockSpec((D, 1), lambda i: (0, 0)),
            pl.BlockSpec((1, 1), lambda i: (0, 0)),
        ],
        out_specs=pl.BlockSpec((B, 1), lambda i: (i, 0)),
        out_shape=jax.ShapeDtypeStruct((2 * B, 1), jnp.float32),
    )(hs, hdn, wsrc, wdst, b, wp, bp)


def _sc_mesh():
    return plsc.VectorSubcoreMesh(core_axis_name="c", subcore_axis_name="s")


def _sc_gather_qkv(qn, kv, src, dst):
    niter = EWH // GC

    @functools.partial(
        pl.kernel,
        out_type=(jax.ShapeDtypeStruct((EH, D), jnp.float32),
                  jax.ShapeDtypeStruct((EH, 2 * D), jnp.float32)),
        mesh=_sc_mesh(),
        scratch_types=[
            pltpu.VMEM((GC,), jnp.int32),
            pltpu.VMEM((GC,), jnp.int32),
            pltpu.VMEM((GC,), jnp.int32),
            pltpu.VMEM((GC,), jnp.int32),
            pltpu.VMEM((GC, D), jnp.float32),
            pltpu.VMEM((GC, D), jnp.float32),
            pltpu.VMEM((GC, 2 * D), jnp.float32),
            pltpu.VMEM((GC, 2 * D), jnp.float32),
            pltpu.SemaphoreType.DMA,
            pltpu.SemaphoreType.DMA,
            pltpu.SemaphoreType.DMA,
            pltpu.SemaphoreType.DMA,
            pltpu.SemaphoreType.DMA,
            pltpu.SemaphoreType.DMA,
        ],
    )
    def k(qn_h, kv_h, src_h, dst_h, qg_h, kvg_h,
          di0, di1, si0, si1, qb0, qb1, kb0, kb1,
          smi0, smi1, sg0, sg1, sw0, sw1):
        wid = lax.axis_index("s") * NC + lax.axis_index("c")
        base = wid * EWH
        dis, sis = (di0, di1), (si0, si1)
        qbs, kbs = (qb0, qb1), (kb0, kb1)
        smis, sgs, sws = (smi0, smi1), (sg0, sg1), (sw0, sw1)
        ih = [None, None]
        g = [None, None]
        w = [None, None]
        for i in range(niter + 1):
            p = i & 1
            q = 1 - p
            if i < niter:
                if g[p] is not None:
                    g[p][0].wait()
                    g[p][1].wait()
                    off2 = base + (i - 2) * GC
                    w[p] = (pltpu.async_copy(qbs[p],
                                             qg_h.at[pl.ds(off2, GC)],
                                             sws[p]),
                            pltpu.async_copy(kbs[p],
                                             kvg_h.at[pl.ds(off2, GC)],
                                             sws[p]))
                    g[p] = None
                off = base + i * GC
                ih[p] = (pltpu.async_copy(dst_h.at[pl.ds(off, GC)], dis[p],
                                          smis[p]),
                         pltpu.async_copy(src_h.at[pl.ds(off, GC)], sis[p],
                                          smis[p]))
            if i > 0:
                ih[q][0].wait()
                ih[q][1].wait()
                if w[q] is not None:
                    w[q][0].wait()
                    w[q][1].wait()
                    w[q] = None
                g[q] = (pltpu.async_copy(qn_h.at[dis[q]], qbs[q], sgs[q]),
                        pltpu.async_copy(kv_h.at[sis[q]], kbs[q], sgs[q]))
        for c in (niter - 2, niter - 1):
            pc = c & 1
            if g[pc] is not None:
                g[pc][0].wait()
                g[pc][1].wait()
                if w[pc] is not None:
                    w[pc][0].wait()
                    w[pc][1].wait()
                offc = base + c * GC
                w[pc] = (pltpu.async_copy(qbs[pc],
                                          qg_h.at[pl.ds(offc, GC)], sws[pc]),
                         pltpu.async_copy(kbs[pc],
                                          kvg_h.at[pl.ds(offc, GC)],
                                          sws[pc]))
        for pc in (0, 1):
            if w[pc] is not None:
                w[pc][0].wait()
                w[pc][1].wait()

    return k(qn, kv, src, dst)


def _sc_scatter_msg(msg, wrow, dst, zacc):
    niter = EWH // GC

    @functools.partial(
        pl.kernel,
        out_type=jax.ShapeDtypeStruct((NC, 2, NP, D), jnp.float32),
        mesh=_sc_mesh(),
        scratch_types=[
            pltpu.VMEM((GC,), jnp.int32),
            pltpu.VMEM((GC,), jnp.int32),
            pltpu.VMEM((GC, D), jnp.float32),
            pltpu.VMEM((GC, D), jnp.float32),
            pltpu.VMEM_SHARED((NP, D), jnp.float32),
            pltpu.SemaphoreType.DMA,
            pltpu.SemaphoreType.DMA,
            pltpu.SemaphoreType.DMA,
            pltpu.SemaphoreType.DMA,
        ],
    )
    def k(msg_h, w_h, dst_h, zacc_h, out_h, di0, di1, mb0, mb1, acc_s,
          sr0, sr1, sa0, sa1):
        cid = lax.axis_index("c")
        sid = lax.axis_index("s")
        wid = sid * NC + cid
        base = wid * EWH
        r0 = sid * ROWS_PER_TILE
        dis, mbs, srs, sas = (di0, di1), (mb0, mb1), (sr0, sr1), (sa0, sa1)

        for phase, src_h in ((0, msg_h), (1, w_h)):
            pltpu.sync_copy(zacc_h.at[pl.ds(r0, ROWS_PER_TILE)],
                            acc_s.at[pl.ds(r0, ROWS_PER_TILE)])
            plsc.subcore_barrier()
            r = [None, None]
            a = [None, None]
            for i in range(niter):
                p = i & 1
                if a[p] is not None:
                    a[p].wait()
                off = base + i * GC
                r[p] = (pltpu.async_copy(dst_h.at[pl.ds(off, GC)], dis[p],
                                         srs[p]),
                        pltpu.async_copy(src_h.at[pl.ds(off, GC)], mbs[p],
                                         srs[p]))
                if i > 0:
                    q = 1 - p
                    r[q][0].wait()
                    r[q][1].wait()
                    a[q] = pltpu.async_copy(mbs[q], acc_s.at[dis[q]],
                                            sas[q], add=True)
            p = (niter - 1) & 1
            r[p][0].wait()
            r[p][1].wait()
            al = pltpu.async_copy(mbs[p], acc_s.at[dis[p]], sas[p], add=True)
            if a[1 - p] is not None:
                a[1 - p].wait()
            al.wait()
            plsc.subcore_barrier()
            pltpu.sync_copy(acc_s.at[pl.ds(r0, ROWS_PER_TILE)],
                            out_h.at[cid, phase, pl.ds(r0, ROWS_PER_TILE)])
            plsc.subcore_barrier()

    return k(msg, wrow, dst, zacc)


def _sc_gather_rows(table, ids):
    n = ids.shape[0]
    c = n // NW

    @functools.partial(
        pl.kernel,
        out_type=jax.ShapeDtypeStruct((n, D), jnp.float32),
        mesh=_sc_mesh(),
        scratch_types=[
            pltpu.VMEM((c,), jnp.int32),
            pltpu.VMEM((c, D), jnp.float32),
            pltpu.SemaphoreType.DMA,
        ],
    )
    def k(tab_h, ids_h, out_h, idx, buf, sem):
        wid = lax.axis_index("s") * NC + lax.axis_index("c")
        base = wid * c
        pltpu.sync_copy(ids_h.at[pl.ds(base, c)], idx)
        pltpu.async_copy(tab_h.at[idx], buf, sem).wait()
        pltpu.sync_copy(buf, out_h.at[pl.ds(base, c)])

    return k(table, ids)


def _pad_rows(x, rows):
    return jnp.pad(x, ((0, rows - x.shape[0]), (0, 0)))


def kernel(node_feat, edge_index, edge_feat, edge_time, src_ids, dst_ids,
           neg_ids, time_w, time_b, Wq, bq, Wk, bk, Wv, bv, Wout, bout,
           gamma, beta, Wsrc, bsrc, Wdst, bdst, Wp, bp):
    f32 = jnp.float32
    srcs = [edge_index[0, s * EH:(s + 1) * EH] for s in range(NSPLIT)]
    dsts = [edge_index[1, s * EH:(s + 1) * EH] for s in range(NSPLIT)]
    efs = [edge_feat[s * EH:(s + 1) * EH] for s in range(NSPLIT)]
    t2s = [edge_time[s * EH:(s + 1) * EH].astype(f32)
           .reshape(EH // BE, BE // 128, 128).swapaxes(1, 2)
           for s in range(NSPLIT)]
    h = _pad_rows(node_feat.astype(f32), NP)
    zacc = jnp.zeros((NP, D), f32)

    for l in range(2):
        wqh = Wq[l][:D]
        wqt = _pad_rows(Wq[l][D:D + DT], DTP)
        te0 = jnp.pad(jnp.cos(time_b[l]), (0, DTP - DT)).reshape(1, DTP)
        bq_row = bq[l].reshape(1, D)
        wkv = jnp.concatenate([Wk[l][:D], Wv[l][:D]], axis=1)
        wt2 = _pad_rows(
            jnp.concatenate([Wk[l][D + DE:], Wv[l][D + DE:]], axis=1), DTP)
        we2 = jnp.concatenate([Wk[l][D:D + DE], Wv[l][D:D + DE]], axis=1)
        bkv = jnp.concatenate([bk[l], bv[l]]).reshape(1, 2 * D)
        tw_row = jnp.pad(time_w[l], (0, DTP - DT)).reshape(1, DTP)
        tb_row = jnp.pad(time_b[l], (0, DTP - DT)).reshape(1, DTP)

        qn, kvn = _node_proj(h, wqh, wqt, te0, bq_row, wkv)
        parts = []
        for s in range(NSPLIT):
            qg, kvg = _sc_gather_qkv(qn, kvn, srcs[s], dsts[s])
            msg, wrow = _edge_msg(t2s[s], efs[s], qg, kvg, tw_row, tb_row,
                                  wt2, we2, bkv)
            parts.append(_sc_scatter_msg(msg, wrow, dsts[s], zacc))
        h = _out_proj(h, parts, Wout[l][:D], Wout[l][D:],
                      bout[l].reshape(1, D), gamma[l].reshape(1, D),
                      beta[l].reshape(1, D))

    ids = jnp.concatenate([src_ids, dst_ids, neg_ids]).astype(jnp.int32)
    g = _sc_gather_rows(h, ids)
    B = src_ids.shape[0]
    hs = g[:B]
    hdn = g[B:].reshape(2, B, D)
    bb = (bsrc + bdst).reshape(1, D)
    return _link_score(hs, hdn, Wsrc, Wdst, bb, Wp, bp.reshape(1, 1))
